# Initial kernel scaffold; baseline (speedup 1.0000x reference)
import jax
import jax.numpy as jnp
from jax.experimental import pallas as pl


def kernel(indices, table):
    raise NotImplementedError("write your pallas kernel here")


# speedup vs baseline: 1.2612x; 1.2612x over previous
import functools

import jax
import jax.numpy as jnp
from jax import lax
from jax.experimental import pallas as pl
from jax.experimental.pallas import tpu as pltpu
from jax.experimental.pallas import tpu_sc as plsc


def _sc_gather(idx, table, n_chunks, chunk, nc, ns):
    nw = nc * ns
    rows_per_w = n_chunks * chunk
    n_total = nw * rows_per_w
    d = table.shape[1]

    mesh = plsc.VectorSubcoreMesh(core_axis_name="c", subcore_axis_name="s")

    @functools.partial(
        pl.kernel,
        out_type=jax.ShapeDtypeStruct((n_total, d), jnp.float32),
        mesh=mesh,
        scratch_types=[
            pltpu.VMEM((n_chunks, chunk), jnp.int32),
            pltpu.VMEM((chunk, d), jnp.float32),
            pltpu.SemaphoreType.DMA,
        ],
    )
    def gather_k(idx_hbm, table_hbm, out_hbm, idx_v, rows_v, gsem):
        wid = lax.axis_index("s") * nc + lax.axis_index("c")
        base = wid * rows_per_w
        pltpu.sync_copy(idx_hbm.at[wid], idx_v)

        def body(j, carry):
            pltpu.async_copy(table_hbm.at[idx_v.at[j]], rows_v, gsem).wait()
            pltpu.sync_copy(rows_v, out_hbm.at[pl.ds(base + j * chunk, chunk)])
            return carry

        lax.fori_loop(0, n_chunks, body, 0)

    return gather_k(idx, table)


def kernel(indices, table):
    b0, b1 = indices.shape
    v, d = table.shape
    n = b0 * b1

    info = plsc.get_sparse_core_info()
    nc, ns = info.num_cores, info.num_subcores
    nw = nc * ns

    chunk = 64
    per_w = n // nw
    n_chunks = per_w // chunk
    assert n == nw * n_chunks * chunk, (n, nw, chunk)

    idx = indices.reshape(nw, n_chunks, chunk).astype(jnp.int32)
    out = _sc_gather(idx, table, n_chunks, chunk, nc, ns)
    return out.reshape(b0, b1, d)


# speedup vs baseline: 1.3059x; 1.0354x over previous
import functools

import jax
import jax.numpy as jnp
from jax import lax
from jax.experimental import pallas as pl
from jax.experimental.pallas import tpu as pltpu
from jax.experimental.pallas import tpu_sc as plsc


def _sc_gather(idx, table, n_chunks, chunk, nc, ns):
    nw = nc * ns
    rows_per_w = n_chunks * chunk
    n_total = nw * rows_per_w
    d = table.shape[1]

    mesh = plsc.VectorSubcoreMesh(core_axis_name="c", subcore_axis_name="s")
    assert n_chunks % 2 == 0 and n_chunks >= 4

    @functools.partial(
        pl.kernel,
        out_type=jax.ShapeDtypeStruct((n_total, d), jnp.float32),
        mesh=mesh,
        scratch_types=[
            pltpu.VMEM((n_chunks, chunk), jnp.int32),
            pltpu.VMEM((chunk, d), jnp.float32),
            pltpu.VMEM((chunk, d), jnp.float32),
            pltpu.SemaphoreType.DMA,
            pltpu.SemaphoreType.DMA,
            pltpu.SemaphoreType.DMA,
            pltpu.SemaphoreType.DMA,
        ],
    )
    def gather_k(idx_hbm, table_hbm, out_hbm, idx_v, buf0, buf1, g0, g1, o0, o1):
        wid = lax.axis_index("s") * nc + lax.axis_index("c")
        base = wid * rows_per_w
        pltpu.sync_copy(idx_hbm.at[wid], idx_v)

        bufs, gs, os_ = (buf0, buf1), (g0, g1), (o0, o1)

        def g_copy(j, b):
            return pltpu.make_async_copy(table_hbm.at[idx_v.at[j]], bufs[b], gs[b])

        def o_copy(j, b):
            return pltpu.make_async_copy(
                bufs[b], out_hbm.at[pl.ds(base + j * chunk, chunk)], os_[b])

        g_copy(0, 0).start()
        g_copy(1, 1).start()

        def body(p, carry):
            j0 = 2 * p
            for b in range(2):
                j = j0 + b
                g_copy(j, b).wait()
                o_copy(j, b).start()
                o_copy(j, b).wait()
                g_copy(j + 2, b).start()
            return carry

        lax.fori_loop(0, n_chunks // 2 - 1, body, 0)
        for b in range(2):
            j = n_chunks - 2 + b
            g_copy(j, b).wait()
            o_copy(j, b).start()
            o_copy(j, b).wait()

    return gather_k(idx, table)


def kernel(indices, table):
    b0, b1 = indices.shape
    v, d = table.shape
    n = b0 * b1

    info = plsc.get_sparse_core_info()
    nc, ns = info.num_cores, info.num_subcores
    nw = nc * ns

    chunk = 64
    per_w = n // nw
    n_chunks = per_w // chunk
    assert n == nw * n_chunks * chunk, (n, nw, chunk)

    idx = indices.reshape(nw, n_chunks, chunk).astype(jnp.int32)
    out = _sc_gather(idx, table, n_chunks, chunk, nc, ns)
    return out.reshape(b0, b1, d)
